# baseline (speedup 1.0000x reference)
import functools

import jax
import jax.numpy as jnp
from jax import lax
from jax.experimental import pallas as pl
from jax.experimental.pallas import tpu as pltpu
from jax.experimental.pallas import tpu_sc as plsc

DIM = 1024
INTER = 512
N_EXPERTS = 8
N_SHARED = 2
T = 2048
TB = 256
S_INTER = INTER * N_SHARED
B = 256
NB = (2 * T) // B + N_EXPERTS
NPAD = NB * B
NC = 2
NS = 16
NW = NC * NS


def _sc_mesh():
    return plsc.VectorSubcoreMesh(
        core_axis_name="c", subcore_axis_name="s", num_cores=NC,
        num_subcores=NS)


def _wid():
    return lax.axis_index("s") * NC + lax.axis_index("c")


def _gate_meta_body(x_ref, gw_ref, p1_ref, p2_ref, wa_ref, wb_ref,
                    eid_ref, nu_ref):
    x = x_ref[...]
    lg = lax.dot_general(gw_ref[...], x, (((1,), (1,)), ((), ())),
                         preferred_element_type=jnp.float32)
    s = jax.nn.sigmoid(lg)
    io8 = lax.broadcasted_iota(jnp.int32, (N_EXPERTS, T), 0)
    m1 = jnp.max(s, axis=0, keepdims=True)
    i1 = jnp.min(jnp.where(s == m1, io8, N_EXPERTS), axis=0, keepdims=True)
    s2 = jnp.where(io8 == i1, -jnp.inf, s)
    m2 = jnp.max(s2, axis=0, keepdims=True)
    i2 = jnp.min(jnp.where(s2 == m2, io8, N_EXPERTS), axis=0, keepdims=True)
    den = m1 + m2
    wa_ref[...] = jnp.reshape(m1 / den, (T,))
    wb_ref[...] = jnp.reshape(m2 / den, (T,))
    sel1 = io8 == i1
    sel2 = io8 == i2
    selm = jnp.where(sel1 | sel2, 1.0, 0.0)
    r = lax.broadcasted_iota(jnp.int32, (T, T), 0)
    c = lax.broadcasted_iota(jnp.int32, (T, T), 1)
    triu = jnp.where(r <= c, 1.0, 0.0)
    csum = lax.dot_general(selm, triu, (((1,), (0,)), ((), ())),
                           preferred_element_type=jnp.float32)
    cnt = csum[:, T - 1:T]
    nblk = jnp.floor((cnt + (B - 1)) * (1.0 / B))
    r8 = lax.broadcasted_iota(jnp.int32, (N_EXPERTS, N_EXPERTS), 0)
    c8 = lax.broadcasted_iota(jnp.int32, (N_EXPERTS, N_EXPERTS), 1)
    strict = jnp.where(r8 > c8, 1.0, 0.0)
    blkbase = lax.dot_general(strict, nblk, (((1,), (0,)), ((), ())),
                              preferred_element_type=jnp.float32)
    pos = blkbase * float(B) + csum - 1.0
    p1 = jnp.sum(jnp.where(sel1, pos, 0.0), axis=0, keepdims=True)
    p2 = jnp.sum(jnp.where(sel2, pos, 0.0), axis=0, keepdims=True)
    p1_ref[...] = jnp.reshape(p1, (T,)).astype(jnp.int32)
    p2_ref[...] = jnp.reshape(p2, (T,)).astype(jnp.int32)
    ii = lax.broadcasted_iota(jnp.int32, (1, 128), 1).astype(jnp.float32)
    acc = jnp.full((1, 128), -1.0, jnp.float32)
    for e in range(N_EXPERTS):
        acc = acc + jnp.where(blkbase[e:e + 1, 0:1] <= ii, 1.0, 0.0)
    eid_ref[...] = jnp.reshape(acc, (128,)).astype(jnp.int32)
    nu = jnp.sum(nblk, axis=0, keepdims=True)
    nu_ref[...] = jnp.reshape(nu, (1,)).astype(jnp.int32)


def _gate_meta(xf, gate_w):
    return pl.pallas_call(
        _gate_meta_body,
        grid=(1,),
        in_specs=[
            pl.BlockSpec((T, DIM), lambda i: (0, 0)),
            pl.BlockSpec((N_EXPERTS, DIM), lambda i: (0, 0)),
        ],
        out_specs=[
            pl.BlockSpec((T,), lambda i: (0,)),
            pl.BlockSpec((T,), lambda i: (0,)),
            pl.BlockSpec((T,), lambda i: (0,)),
            pl.BlockSpec((T,), lambda i: (0,)),
            pl.BlockSpec((128,), lambda i: (0,)),
            pl.BlockSpec((1,), lambda i: (0,)),
        ],
        out_shape=[
            jax.ShapeDtypeStruct((T,), jnp.int32),
            jax.ShapeDtypeStruct((T,), jnp.int32),
            jax.ShapeDtypeStruct((T,), jnp.float32),
            jax.ShapeDtypeStruct((T,), jnp.float32),
            jax.ShapeDtypeStruct((128,), jnp.int32),
            jax.ShapeDtypeStruct((1,), jnp.int32),
        ],
    )(xf, gate_w)


def _shared_body(x_ref, sw1_ref, sw2_ref, sw3_ref, ys_ref, xb_ref):
    xb = x_ref[...].astype(jnp.bfloat16)
    xb_ref[...] = xb
    w1 = sw1_ref[...].astype(jnp.bfloat16)
    w2 = sw2_ref[...].astype(jnp.bfloat16)
    w3 = sw3_ref[...].astype(jnp.bfloat16)
    h1 = lax.dot_general(xb, w1, (((1,), (1,)), ((), ())),
                         preferred_element_type=jnp.float32)
    h3 = lax.dot_general(xb, w3, (((1,), (1,)), ((), ())),
                         preferred_element_type=jnp.float32)
    h = (h1 * jax.nn.sigmoid(h1) * h3).astype(jnp.bfloat16)
    ys_ref[...] = lax.dot_general(h, w2, (((1,), (1,)), ((), ())),
                                  preferred_element_type=jnp.float32)


def _shared(xf, sw1, sw2, sw3):
    nb = T // TB
    return pl.pallas_call(
        _shared_body,
        grid=(nb,),
        in_specs=[
            pl.BlockSpec((TB, DIM), lambda i: (i, 0)),
            pl.BlockSpec((S_INTER, DIM), lambda i: (0, 0)),
            pl.BlockSpec((DIM, S_INTER), lambda i: (0, 0)),
            pl.BlockSpec((S_INTER, DIM), lambda i: (0, 0)),
        ],
        out_specs=[
            pl.BlockSpec((TB, DIM), lambda i: (i, 0)),
            pl.BlockSpec((TB, DIM), lambda i: (i, 0)),
        ],
        out_shape=[
            jax.ShapeDtypeStruct((T, DIM), jnp.float32),
            jax.ShapeDtypeStruct((T, DIM), jnp.bfloat16),
        ],
    )(xf, sw1, sw2, sw3)


def _sc_scatter_routing(pos1, pos2, wa, wb):
    def body(p1_hbm, p2_hbm, wa_hbm, wb_hbm, tok_hbm, wrow_hbm,
             tok_v, wrow_v, pos_v, w_v):
        @pl.when(_wid() == 0)
        def _():
            def init(i, carry):
                tok_v[pl.ds(i * 16, 16)] = jnp.zeros((16,), jnp.int32)
                wrow_v[pl.ds(i * 16, 16)] = jnp.zeros((16,), jnp.float32)
                return carry
            lax.fori_loop(0, NPAD // 16, init, 0)
            for p_hbm, wx_hbm in ((p1_hbm, wa_hbm), (p2_hbm, wb_hbm)):
                pltpu.sync_copy(p_hbm, pos_v)
                pltpu.sync_copy(wx_hbm, w_v)

                def step(i, carry):
                    idx = pos_v[pl.ds(i * 16, 16)]
                    tvals = lax.iota(jnp.int32, 16) + i * 16
                    plsc.store_scatter(tok_v, [idx], tvals)
                    wv = w_v[pl.ds(i * 16, 16)]
                    plsc.store_scatter(wrow_v, [idx], wv)
                    return carry
                lax.fori_loop(0, T // 16, step, 0)
            pltpu.sync_copy(tok_v, tok_hbm)
            pltpu.sync_copy(wrow_v, wrow_hbm)

    fn = pl.kernel(
        body,
        out_type=[
            jax.ShapeDtypeStruct((NPAD,), jnp.int32),
            jax.ShapeDtypeStruct((NPAD,), jnp.float32),
        ],
        mesh=_sc_mesh(),
        scratch_types=[
            pltpu.VMEM((NPAD,), jnp.int32),
            pltpu.VMEM((NPAD,), jnp.float32),
            pltpu.VMEM((T,), jnp.int32),
            pltpu.VMEM((T,), jnp.float32),
        ],
        compiler_params=pltpu.CompilerParams(needs_layout_passes=False),
    )
    return fn(pos1, pos2, wa, wb)


def _mlp_blk(xg, w1_ref, w2_ref, w3_ref):
    w1 = w1_ref[0].astype(jnp.bfloat16)
    w2 = w2_ref[0].astype(jnp.bfloat16)
    w3 = w3_ref[0].astype(jnp.bfloat16)
    h1 = lax.dot_general(xg, w1, (((1,), (1,)), ((), ())),
                         preferred_element_type=jnp.float32)
    h3 = lax.dot_general(xg, w3, (((1,), (1,)), ((), ())),
                         preferred_element_type=jnp.float32)
    h = (h1 * jax.nn.sigmoid(h1) * h3).astype(jnp.bfloat16)
    return lax.dot_general(h, w2, (((1,), (1,)), ((), ())),
                           preferred_element_type=jnp.float32
                           ).astype(jnp.bfloat16)


def _gemm_body(eid_ref, nu_ref, tok_ref, wrow_ref, xb_ref, ys_ref,
               ew1a_ref, ew2a_ref, ew3a_ref, ew1b_ref, ew2b_ref, ew3b_ref,
               o_ref):
    i = pl.program_id(0)

    @pl.when(i == 0)
    def _():
        o_ref[...] = ys_ref[...]

    @pl.when(2 * i < nu_ref[0])
    def _():
        tok = jnp.reshape(tok_ref[...], (1, 2 * B))
        wrow = jnp.reshape(wrow_ref[...], (1, 2 * B))
        tio = lax.broadcasted_iota(jnp.int32, (T, 2 * B), 0)
        ohm = tio == tok
        oh = ohm.astype(jnp.bfloat16)
        ohw = jnp.where(ohm, wrow, 0.0).astype(jnp.bfloat16)
        xg2 = lax.dot_general(oh, xb_ref[...], (((0,), (0,)), ((), ())),
                              preferred_element_type=jnp.float32
                              ).astype(jnp.bfloat16)
        ya = _mlp_blk(xg2[:B], ew1a_ref, ew2a_ref, ew3a_ref)
        yb = _mlp_blk(xg2[B:], ew1b_ref, ew2b_ref, ew3b_ref)
        ycat = jnp.concatenate([ya, yb], axis=0)
        o_ref[...] += lax.dot_general(ohw, ycat, (((1,), (0,)), ((), ())),
                                      preferred_element_type=jnp.float32)


def _gemm(eid, nused, tok, wrow, xb, ys, ew1, ew2, ew3):
    ew_spec_a = lambda i, eid, nu: (eid[2 * i], 0, 0)
    ew_spec_b = lambda i, eid, nu: (eid[2 * i + 1], 0, 0)
    return pl.pallas_call(
        _gemm_body,
        grid_spec=pltpu.PrefetchScalarGridSpec(
            num_scalar_prefetch=2,
            grid=(NB // 2,),
            in_specs=[
                pl.BlockSpec((2 * B,), lambda i, eid, nu: (i,)),
                pl.BlockSpec((2 * B,), lambda i, eid, nu: (i,)),
                pl.BlockSpec((T, DIM), lambda i, eid, nu: (0, 0)),
                pl.BlockSpec((T, DIM), lambda i, eid, nu: (0, 0)),
                pl.BlockSpec((1, INTER, DIM), ew_spec_a),
                pl.BlockSpec((1, DIM, INTER), ew_spec_a),
                pl.BlockSpec((1, INTER, DIM), ew_spec_a),
                pl.BlockSpec((1, INTER, DIM), ew_spec_b),
                pl.BlockSpec((1, DIM, INTER), ew_spec_b),
                pl.BlockSpec((1, INTER, DIM), ew_spec_b),
            ],
            out_specs=pl.BlockSpec((T, DIM), lambda i, eid, nu: (0, 0)),
        ),
        out_shape=jax.ShapeDtypeStruct((T, DIM), jnp.float32),
    )(eid, nused, tok, wrow, xb, ys, ew1, ew2, ew3, ew1, ew2, ew3)


def kernel(x, gate_w, ew1, ew2, ew3, sw1, sw2, sw3):
    shape = x.shape
    xf = x.reshape(-1, DIM)

    pos1, pos2, wa, wb, eid, nused = _gate_meta(xf, gate_w)
    tok, wrow = _sc_scatter_routing(pos1, pos2, wa, wb)
    ys, xb = _shared(xf, sw1, sw2, sw3)
    y = _gemm(eid, nused, tok, wrow, xb, ys, ew1, ew2, ew3)
    return y.reshape(shape)


# speedup vs baseline: 1.0328x; 1.0328x over previous
import functools

import jax
import jax.numpy as jnp
from jax import lax
from jax.experimental import pallas as pl
from jax.experimental.pallas import tpu as pltpu
from jax.experimental.pallas import tpu_sc as plsc

DIM = 1024
INTER = 512
N_EXPERTS = 8
N_SHARED = 2
T = 2048
TB = 512
S_INTER = INTER * N_SHARED
B = 256
NB = (2 * T) // B + N_EXPERTS
NPAD = NB * B
NC = 2
NS = 16
NW = NC * NS


def _sc_mesh():
    return plsc.VectorSubcoreMesh(
        core_axis_name="c", subcore_axis_name="s", num_cores=NC,
        num_subcores=NS)


def _wid():
    return lax.axis_index("s") * NC + lax.axis_index("c")


def _gate_meta_body(x_ref, gw_ref, p1_ref, p2_ref, wa_ref, wb_ref,
                    eid_ref, nu_ref):
    x = x_ref[...]
    lg = lax.dot_general(gw_ref[...], x, (((1,), (1,)), ((), ())),
                         preferred_element_type=jnp.float32)
    s = jax.nn.sigmoid(lg)
    io8 = lax.broadcasted_iota(jnp.int32, (N_EXPERTS, T), 0)
    m1 = jnp.max(s, axis=0, keepdims=True)
    i1 = jnp.min(jnp.where(s == m1, io8, N_EXPERTS), axis=0, keepdims=True)
    s2 = jnp.where(io8 == i1, -jnp.inf, s)
    m2 = jnp.max(s2, axis=0, keepdims=True)
    i2 = jnp.min(jnp.where(s2 == m2, io8, N_EXPERTS), axis=0, keepdims=True)
    den = m1 + m2
    wa_ref[...] = jnp.reshape(m1 / den, (T,))
    wb_ref[...] = jnp.reshape(m2 / den, (T,))
    sel1 = io8 == i1
    sel2 = io8 == i2
    selm = jnp.where(sel1 | sel2, 1.0, 0.0)
    r = lax.broadcasted_iota(jnp.int32, (T, T), 0)
    c = lax.broadcasted_iota(jnp.int32, (T, T), 1)
    triu = jnp.where(r <= c, 1.0, 0.0)
    csum = lax.dot_general(selm, triu, (((1,), (0,)), ((), ())),
                           preferred_element_type=jnp.float32)
    cnt = csum[:, T - 1:T]
    nblk = jnp.floor((cnt + (B - 1)) * (1.0 / B))
    r8 = lax.broadcasted_iota(jnp.int32, (N_EXPERTS, N_EXPERTS), 0)
    c8 = lax.broadcasted_iota(jnp.int32, (N_EXPERTS, N_EXPERTS), 1)
    strict = jnp.where(r8 > c8, 1.0, 0.0)
    blkbase = lax.dot_general(strict, nblk, (((1,), (0,)), ((), ())),
                              preferred_element_type=jnp.float32)
    pos = blkbase * float(B) + csum - 1.0
    p1 = jnp.sum(jnp.where(sel1, pos, 0.0), axis=0, keepdims=True)
    p2 = jnp.sum(jnp.where(sel2, pos, 0.0), axis=0, keepdims=True)
    p1_ref[...] = jnp.reshape(p1, (T,)).astype(jnp.int32)
    p2_ref[...] = jnp.reshape(p2, (T,)).astype(jnp.int32)
    ii = lax.broadcasted_iota(jnp.int32, (1, 128), 1).astype(jnp.float32)
    acc = jnp.full((1, 128), -1.0, jnp.float32)
    for e in range(N_EXPERTS):
        acc = acc + jnp.where(blkbase[e:e + 1, 0:1] <= ii, 1.0, 0.0)
    eid_ref[...] = jnp.reshape(acc, (128,)).astype(jnp.int32)
    nu = jnp.sum(nblk, axis=0, keepdims=True)
    nu_ref[...] = jnp.reshape(nu, (1,)).astype(jnp.int32)


def _gate_meta(xf, gate_w):
    return pl.pallas_call(
        _gate_meta_body,
        grid=(1,),
        in_specs=[
            pl.BlockSpec((T, DIM), lambda i: (0, 0)),
            pl.BlockSpec((N_EXPERTS, DIM), lambda i: (0, 0)),
        ],
        out_specs=[
            pl.BlockSpec((T,), lambda i: (0,)),
            pl.BlockSpec((T,), lambda i: (0,)),
            pl.BlockSpec((T,), lambda i: (0,)),
            pl.BlockSpec((T,), lambda i: (0,)),
            pl.BlockSpec((128,), lambda i: (0,)),
            pl.BlockSpec((1,), lambda i: (0,)),
        ],
        out_shape=[
            jax.ShapeDtypeStruct((T,), jnp.int32),
            jax.ShapeDtypeStruct((T,), jnp.int32),
            jax.ShapeDtypeStruct((T,), jnp.float32),
            jax.ShapeDtypeStruct((T,), jnp.float32),
            jax.ShapeDtypeStruct((128,), jnp.int32),
            jax.ShapeDtypeStruct((1,), jnp.int32),
        ],
    )(xf, gate_w)


def _shared_body(x_ref, sw1_ref, sw2_ref, sw3_ref, ys_ref, xb_ref):
    xb = x_ref[...].astype(jnp.bfloat16)
    xb_ref[...] = xb
    w1 = sw1_ref[...].astype(jnp.bfloat16)
    w2 = sw2_ref[...].astype(jnp.bfloat16)
    w3 = sw3_ref[...].astype(jnp.bfloat16)
    h1 = lax.dot_general(xb, w1, (((1,), (1,)), ((), ())),
                         preferred_element_type=jnp.float32)
    h3 = lax.dot_general(xb, w3, (((1,), (1,)), ((), ())),
                         preferred_element_type=jnp.float32)
    h = (h1 * jax.nn.sigmoid(h1) * h3).astype(jnp.bfloat16)
    ys_ref[...] = lax.dot_general(h, w2, (((1,), (1,)), ((), ())),
                                  preferred_element_type=jnp.float32)


def _shared(xf, sw1, sw2, sw3):
    nb = T // TB
    return pl.pallas_call(
        _shared_body,
        grid=(nb,),
        in_specs=[
            pl.BlockSpec((TB, DIM), lambda i: (i, 0)),
            pl.BlockSpec((S_INTER, DIM), lambda i: (0, 0)),
            pl.BlockSpec((DIM, S_INTER), lambda i: (0, 0)),
            pl.BlockSpec((S_INTER, DIM), lambda i: (0, 0)),
        ],
        out_specs=[
            pl.BlockSpec((TB, DIM), lambda i: (i, 0)),
            pl.BlockSpec((TB, DIM), lambda i: (i, 0)),
        ],
        out_shape=[
            jax.ShapeDtypeStruct((T, DIM), jnp.float32),
            jax.ShapeDtypeStruct((T, DIM), jnp.bfloat16),
        ],
    )(xf, sw1, sw2, sw3)


def _sc_scatter_routing(pos1, pos2, wa, wb):
    def body(p1_hbm, p2_hbm, wa_hbm, wb_hbm, tok_hbm, wrow_hbm,
             tok_v, wrow_v, pos_v, w_v):
        @pl.when(_wid() == 0)
        def _():
            def init(i, carry):
                tok_v[pl.ds(i * 16, 16)] = jnp.zeros((16,), jnp.int32)
                wrow_v[pl.ds(i * 16, 16)] = jnp.zeros((16,), jnp.float32)
                return carry
            lax.fori_loop(0, NPAD // 16, init, 0)
            for p_hbm, wx_hbm in ((p1_hbm, wa_hbm), (p2_hbm, wb_hbm)):
                pltpu.sync_copy(p_hbm, pos_v)
                pltpu.sync_copy(wx_hbm, w_v)

                def step(i, carry):
                    idx = pos_v[pl.ds(i * 16, 16)]
                    tvals = lax.iota(jnp.int32, 16) + i * 16
                    plsc.store_scatter(tok_v, [idx], tvals)
                    wv = w_v[pl.ds(i * 16, 16)]
                    plsc.store_scatter(wrow_v, [idx], wv)
                    return carry
                lax.fori_loop(0, T // 16, step, 0)
            pltpu.sync_copy(tok_v, tok_hbm)
            pltpu.sync_copy(wrow_v, wrow_hbm)

    fn = pl.kernel(
        body,
        out_type=[
            jax.ShapeDtypeStruct((NPAD,), jnp.int32),
            jax.ShapeDtypeStruct((NPAD,), jnp.float32),
        ],
        mesh=_sc_mesh(),
        scratch_types=[
            pltpu.VMEM((NPAD,), jnp.int32),
            pltpu.VMEM((NPAD,), jnp.float32),
            pltpu.VMEM((T,), jnp.int32),
            pltpu.VMEM((T,), jnp.float32),
        ],
        compiler_params=pltpu.CompilerParams(needs_layout_passes=False),
    )
    return fn(pos1, pos2, wa, wb)


def _gemm_body(eid_ref, nu_ref, tok_ref, wrow_ref, xb_ref, ys_ref,
               ew1_ref, ew2_ref, ew3_ref, o_ref):
    i = pl.program_id(0)

    @pl.when(i == 0)
    def _():
        o_ref[...] = ys_ref[...]

    @pl.when(i < nu_ref[0])
    def _():
        tok = jnp.reshape(tok_ref[...], (1, B))
        wrow = jnp.reshape(wrow_ref[...], (1, B))
        tio = lax.broadcasted_iota(jnp.int32, (T, B), 0)
        ohm = tio == tok
        oh = ohm.astype(jnp.bfloat16)
        ohw = jnp.where(ohm, wrow, 0.0).astype(jnp.bfloat16)
        xg = lax.dot_general(oh, xb_ref[...], (((0,), (0,)), ((), ())),
                             preferred_element_type=jnp.float32
                             ).astype(jnp.bfloat16)
        w1 = ew1_ref[0].astype(jnp.bfloat16)
        w2 = ew2_ref[0].astype(jnp.bfloat16)
        w3 = ew3_ref[0].astype(jnp.bfloat16)
        h1 = lax.dot_general(xg, w1, (((1,), (1,)), ((), ())),
                             preferred_element_type=jnp.float32)
        h3 = lax.dot_general(xg, w3, (((1,), (1,)), ((), ())),
                             preferred_element_type=jnp.float32)
        h = (h1 * jax.nn.sigmoid(h1) * h3).astype(jnp.bfloat16)
        y = lax.dot_general(h, w2, (((1,), (1,)), ((), ())),
                            preferred_element_type=jnp.float32
                            ).astype(jnp.bfloat16)
        o_ref[...] += lax.dot_general(ohw, y, (((1,), (0,)), ((), ())),
                                      preferred_element_type=jnp.float32)


def _gemm(eid, nused, tok, wrow, xb, ys, ew1, ew2, ew3):
    return pl.pallas_call(
        _gemm_body,
        grid_spec=pltpu.PrefetchScalarGridSpec(
            num_scalar_prefetch=2,
            grid=(NB,),
            in_specs=[
                pl.BlockSpec((B,), lambda i, eid, nu: (i,)),
                pl.BlockSpec((B,), lambda i, eid, nu: (i,)),
                pl.BlockSpec((T, DIM), lambda i, eid, nu: (0, 0)),
                pl.BlockSpec((T, DIM), lambda i, eid, nu: (0, 0)),
                pl.BlockSpec((1, INTER, DIM), lambda i, eid, nu: (eid[i], 0, 0)),
                pl.BlockSpec((1, DIM, INTER), lambda i, eid, nu: (eid[i], 0, 0)),
                pl.BlockSpec((1, INTER, DIM), lambda i, eid, nu: (eid[i], 0, 0)),
            ],
            out_specs=pl.BlockSpec((T, DIM), lambda i, eid, nu: (0, 0)),
        ),
        out_shape=jax.ShapeDtypeStruct((T, DIM), jnp.float32),
    )(eid, nused, tok, wrow, xb, ys, ew1, ew2, ew3)


def kernel(x, gate_w, ew1, ew2, ew3, sw1, sw2, sw3):
    shape = x.shape
    xf = x.reshape(-1, DIM)

    pos1, pos2, wa, wb, eid, nused = _gate_meta(xf, gate_w)
    tok, wrow = _sc_scatter_routing(pos1, pos2, wa, wb)
    ys, xb = _shared(xf, sw1, sw2, sw3)
    y = _gemm(eid, nused, tok, wrow, xb, ys, ew1, ew2, ew3)
    return y.reshape(shape)


# speedup vs baseline: 1.0354x; 1.0025x over previous
import functools

import jax
import jax.numpy as jnp
from jax import lax
from jax.experimental import pallas as pl
from jax.experimental.pallas import tpu as pltpu
from jax.experimental.pallas import tpu_sc as plsc

DIM = 1024
INTER = 512
N_EXPERTS = 8
N_SHARED = 2
T = 2048
TB = 512
S_INTER = INTER * N_SHARED
B = 256
NB = (2 * T) // B + N_EXPERTS
NPAD = NB * B
NC = 2
NS = 16
NW = NC * NS


def _sc_mesh():
    return plsc.VectorSubcoreMesh(
        core_axis_name="c", subcore_axis_name="s", num_cores=NC,
        num_subcores=NS)


def _wid():
    return lax.axis_index("s") * NC + lax.axis_index("c")


def _gate_meta_body(x_ref, gw_ref, p1_ref, p2_ref, wa_ref, wb_ref,
                    eid_ref, nu_ref):
    x = x_ref[...]
    lg = lax.dot_general(gw_ref[...], x, (((1,), (1,)), ((), ())),
                         preferred_element_type=jnp.float32)
    s = jax.nn.sigmoid(lg)
    io8 = lax.broadcasted_iota(jnp.int32, (N_EXPERTS, T), 0)
    m1 = jnp.max(s, axis=0, keepdims=True)
    i1 = jnp.min(jnp.where(s == m1, io8, N_EXPERTS), axis=0, keepdims=True)
    s2 = jnp.where(io8 == i1, -jnp.inf, s)
    m2 = jnp.max(s2, axis=0, keepdims=True)
    i2 = jnp.min(jnp.where(s2 == m2, io8, N_EXPERTS), axis=0, keepdims=True)
    den = m1 + m2
    wa_ref[...] = jnp.reshape(m1 / den, (T,))
    wb_ref[...] = jnp.reshape(m2 / den, (T,))
    sel1 = io8 == i1
    sel2 = io8 == i2
    selm = jnp.where(sel1 | sel2, 1.0, 0.0)
    r = lax.broadcasted_iota(jnp.int32, (T, T), 0)
    c = lax.broadcasted_iota(jnp.int32, (T, T), 1)
    triu = jnp.where(r <= c, 1.0, 0.0)
    csum = lax.dot_general(selm, triu, (((1,), (0,)), ((), ())),
                           preferred_element_type=jnp.float32)
    cnt = csum[:, T - 1:T]
    nblk = jnp.floor((cnt + (B - 1)) * (1.0 / B))
    r8 = lax.broadcasted_iota(jnp.int32, (N_EXPERTS, N_EXPERTS), 0)
    c8 = lax.broadcasted_iota(jnp.int32, (N_EXPERTS, N_EXPERTS), 1)
    strict = jnp.where(r8 > c8, 1.0, 0.0)
    blkbase = lax.dot_general(strict, nblk, (((1,), (0,)), ((), ())),
                              preferred_element_type=jnp.float32)
    pos = blkbase * float(B) + csum - 1.0
    p1 = jnp.sum(jnp.where(sel1, pos, 0.0), axis=0, keepdims=True)
    p2 = jnp.sum(jnp.where(sel2, pos, 0.0), axis=0, keepdims=True)
    p1_ref[...] = jnp.reshape(p1, (T,)).astype(jnp.int32)
    p2_ref[...] = jnp.reshape(p2, (T,)).astype(jnp.int32)
    ii = lax.broadcasted_iota(jnp.int32, (1, 128), 1).astype(jnp.float32)
    acc = jnp.full((1, 128), -1.0, jnp.float32)
    for e in range(N_EXPERTS):
        acc = acc + jnp.where(blkbase[e:e + 1, 0:1] <= ii, 1.0, 0.0)
    eid_ref[...] = jnp.reshape(acc, (128,)).astype(jnp.int32)
    nu = jnp.sum(nblk, axis=0, keepdims=True)
    nu_ref[...] = jnp.reshape(nu, (1,)).astype(jnp.int32)


def _gate_meta(xf, gate_w):
    return pl.pallas_call(
        _gate_meta_body,
        grid=(1,),
        in_specs=[
            pl.BlockSpec((T, DIM), lambda i: (0, 0)),
            pl.BlockSpec((N_EXPERTS, DIM), lambda i: (0, 0)),
        ],
        out_specs=[
            pl.BlockSpec((T,), lambda i: (0,)),
            pl.BlockSpec((T,), lambda i: (0,)),
            pl.BlockSpec((T,), lambda i: (0,)),
            pl.BlockSpec((T,), lambda i: (0,)),
            pl.BlockSpec((128,), lambda i: (0,)),
            pl.BlockSpec((1,), lambda i: (0,)),
        ],
        out_shape=[
            jax.ShapeDtypeStruct((T,), jnp.int32),
            jax.ShapeDtypeStruct((T,), jnp.int32),
            jax.ShapeDtypeStruct((T,), jnp.float32),
            jax.ShapeDtypeStruct((T,), jnp.float32),
            jax.ShapeDtypeStruct((128,), jnp.int32),
            jax.ShapeDtypeStruct((1,), jnp.int32),
        ],
    )(xf, gate_w)


def _shared_body(x_ref, sw1_ref, sw2_ref, sw3_ref, ys_ref, xb_ref):
    xb = x_ref[...].astype(jnp.bfloat16)
    xb_ref[...] = xb
    w1 = sw1_ref[...].astype(jnp.bfloat16)
    w2 = sw2_ref[...].astype(jnp.bfloat16)
    w3 = sw3_ref[...].astype(jnp.bfloat16)
    h1 = lax.dot_general(xb, w1, (((1,), (1,)), ((), ())),
                         preferred_element_type=jnp.float32)
    h3 = lax.dot_general(xb, w3, (((1,), (1,)), ((), ())),
                         preferred_element_type=jnp.float32)
    h = (h1 * jax.nn.sigmoid(h1) * h3).astype(jnp.bfloat16)
    ys_ref[...] = lax.dot_general(h, w2, (((1,), (1,)), ((), ())),
                                  preferred_element_type=jnp.float32)


def _shared(xf, sw1, sw2, sw3):
    nb = T // TB
    return pl.pallas_call(
        _shared_body,
        grid=(nb,),
        in_specs=[
            pl.BlockSpec((TB, DIM), lambda i: (i, 0)),
            pl.BlockSpec((S_INTER, DIM), lambda i: (0, 0)),
            pl.BlockSpec((DIM, S_INTER), lambda i: (0, 0)),
            pl.BlockSpec((S_INTER, DIM), lambda i: (0, 0)),
        ],
        out_specs=[
            pl.BlockSpec((TB, DIM), lambda i: (i, 0)),
            pl.BlockSpec((TB, DIM), lambda i: (i, 0)),
        ],
        out_shape=[
            jax.ShapeDtypeStruct((T, DIM), jnp.float32),
            jax.ShapeDtypeStruct((T, DIM), jnp.bfloat16),
        ],
    )(xf, sw1, sw2, sw3)


def _sc_scatter_routing(pos1, pos2, wa, wb):
    def body(p1_hbm, p2_hbm, wa_hbm, wb_hbm, tok_hbm, wrow_hbm,
             tok_v, wrow_v, pos_v, w_v):
        @pl.when(_wid() == 0)
        def _():
            def init(i, carry):
                tok_v[pl.ds(i * 16, 16)] = jnp.zeros((16,), jnp.int32)
                wrow_v[pl.ds(i * 16, 16)] = jnp.zeros((16,), jnp.float32)
                return carry
            lax.fori_loop(0, NPAD // 16, init, 0)
            for p_hbm, wx_hbm in ((p1_hbm, wa_hbm), (p2_hbm, wb_hbm)):
                pltpu.sync_copy(p_hbm, pos_v)
                pltpu.sync_copy(wx_hbm, w_v)

                def step(i, carry):
                    idx = pos_v[pl.ds(i * 16, 16)]
                    tvals = lax.iota(jnp.int32, 16) + i * 16
                    plsc.store_scatter(tok_v, [idx], tvals)
                    wv = w_v[pl.ds(i * 16, 16)]
                    plsc.store_scatter(wrow_v, [idx], wv)
                    return carry
                lax.fori_loop(0, T // 16, step, 0)
            pltpu.sync_copy(tok_v, tok_hbm)
            pltpu.sync_copy(wrow_v, wrow_hbm)

    fn = pl.kernel(
        body,
        out_type=[
            jax.ShapeDtypeStruct((NPAD,), jnp.int32),
            jax.ShapeDtypeStruct((NPAD,), jnp.float32),
        ],
        mesh=_sc_mesh(),
        scratch_types=[
            pltpu.VMEM((NPAD,), jnp.int32),
            pltpu.VMEM((NPAD,), jnp.float32),
            pltpu.VMEM((T,), jnp.int32),
            pltpu.VMEM((T,), jnp.float32),
        ],
        compiler_params=pltpu.CompilerParams(needs_layout_passes=False,
                                             skip_device_barrier=True),
    )
    return fn(pos1, pos2, wa, wb)


def _gemm_body(eid_ref, nu_ref, tok_ref, wrow_ref, xb_ref, ys_ref,
               ew1_ref, ew2_ref, ew3_ref, o_ref):
    i = pl.program_id(0)

    @pl.when(i == 0)
    def _():
        o_ref[...] = ys_ref[...]

    @pl.when(i < nu_ref[0])
    def _():
        tok = jnp.reshape(tok_ref[...], (1, B))
        wrow = jnp.reshape(wrow_ref[...], (1, B))
        tio = lax.broadcasted_iota(jnp.int32, (T, B), 0)
        ohm = tio == tok
        oh = ohm.astype(jnp.bfloat16)
        ohw = jnp.where(ohm, wrow, 0.0).astype(jnp.bfloat16)
        xg = lax.dot_general(oh, xb_ref[...], (((0,), (0,)), ((), ())),
                             preferred_element_type=jnp.float32
                             ).astype(jnp.bfloat16)
        w1 = ew1_ref[0].astype(jnp.bfloat16)
        w2 = ew2_ref[0].astype(jnp.bfloat16)
        w3 = ew3_ref[0].astype(jnp.bfloat16)
        h1 = lax.dot_general(xg, w1, (((1,), (1,)), ((), ())),
                             preferred_element_type=jnp.float32)
        h3 = lax.dot_general(xg, w3, (((1,), (1,)), ((), ())),
                             preferred_element_type=jnp.float32)
        h = (h1 * jax.nn.sigmoid(h1) * h3).astype(jnp.bfloat16)
        y = lax.dot_general(h, w2, (((1,), (1,)), ((), ())),
                            preferred_element_type=jnp.float32
                            ).astype(jnp.bfloat16)
        o_ref[...] += lax.dot_general(ohw, y, (((1,), (0,)), ((), ())),
                                      preferred_element_type=jnp.float32)


def _gemm(eid, nused, tok, wrow, xb, ys, ew1, ew2, ew3):
    return pl.pallas_call(
        _gemm_body,
        grid_spec=pltpu.PrefetchScalarGridSpec(
            num_scalar_prefetch=2,
            grid=(NB,),
            in_specs=[
                pl.BlockSpec((B,), lambda i, eid, nu: (i,)),
                pl.BlockSpec((B,), lambda i, eid, nu: (i,)),
                pl.BlockSpec((T, DIM), lambda i, eid, nu: (0, 0)),
                pl.BlockSpec((T, DIM), lambda i, eid, nu: (0, 0)),
                pl.BlockSpec((1, INTER, DIM), lambda i, eid, nu: (eid[i], 0, 0)),
                pl.BlockSpec((1, DIM, INTER), lambda i, eid, nu: (eid[i], 0, 0)),
                pl.BlockSpec((1, INTER, DIM), lambda i, eid, nu: (eid[i], 0, 0)),
            ],
            out_specs=pl.BlockSpec((T, DIM), lambda i, eid, nu: (0, 0)),
        ),
        out_shape=jax.ShapeDtypeStruct((T, DIM), jnp.float32),
    )(eid, nused, tok, wrow, xb, ys, ew1, ew2, ew3)


def kernel(x, gate_w, ew1, ew2, ew3, sw1, sw2, sw3):
    shape = x.shape
    xf = x.reshape(-1, DIM)

    pos1, pos2, wa, wb, eid, nused = _gate_meta(xf, gate_w)
    tok, wrow = _sc_scatter_routing(pos1, pos2, wa, wb)
    ys, xb = _shared(xf, sw1, sw2, sw3)
    y = _gemm(eid, nused, tok, wrow, xb, ys, ew1, ew2, ew3)
    return y.reshape(shape)


# speedup vs baseline: 1.0377x; 1.0022x over previous
import functools

import jax
import jax.numpy as jnp
from jax import lax
from jax.experimental import pallas as pl
from jax.experimental.pallas import tpu as pltpu
from jax.experimental.pallas import tpu_sc as plsc

DIM = 1024
INTER = 512
N_EXPERTS = 8
N_SHARED = 2
T = 2048
TB = 512
S_INTER = INTER * N_SHARED
B = 256
NB = (2 * T) // B + N_EXPERTS
NPAD = NB * B
NC = 2
NS = 16
NW = NC * NS


def _sc_mesh():
    return plsc.VectorSubcoreMesh(
        core_axis_name="c", subcore_axis_name="s", num_cores=NC,
        num_subcores=NS)


def _wid():
    return lax.axis_index("s") * NC + lax.axis_index("c")


def _gate_meta_body(x_ref, gw_ref, p1_ref, p2_ref, wa_ref, wb_ref,
                    eid_ref, nu_ref):
    x = x_ref[...]
    lg = lax.dot_general(gw_ref[...], x, (((1,), (1,)), ((), ())),
                         preferred_element_type=jnp.float32)
    s = jax.nn.sigmoid(lg)
    io8 = lax.broadcasted_iota(jnp.int32, (N_EXPERTS, T), 0)
    m1 = jnp.max(s, axis=0, keepdims=True)
    i1 = jnp.min(jnp.where(s == m1, io8, N_EXPERTS), axis=0, keepdims=True)
    s2 = jnp.where(io8 == i1, -jnp.inf, s)
    m2 = jnp.max(s2, axis=0, keepdims=True)
    i2 = jnp.min(jnp.where(s2 == m2, io8, N_EXPERTS), axis=0, keepdims=True)
    den = m1 + m2
    wa_ref[...] = jnp.reshape(m1 / den, (T,))
    wb_ref[...] = jnp.reshape(m2 / den, (T,))
    sel1 = io8 == i1
    sel2 = io8 == i2
    selm = jnp.where(sel1 | sel2, 1.0, 0.0)
    r = lax.broadcasted_iota(jnp.int32, (T, T), 0)
    c = lax.broadcasted_iota(jnp.int32, (T, T), 1)
    triu = jnp.where(r <= c, 1.0, 0.0)
    csum = lax.dot_general(selm, triu, (((1,), (0,)), ((), ())),
                           preferred_element_type=jnp.float32)
    cnt = csum[:, T - 1:T]
    nblk = jnp.floor((cnt + (B - 1)) * (1.0 / B))
    r8 = lax.broadcasted_iota(jnp.int32, (N_EXPERTS, N_EXPERTS), 0)
    c8 = lax.broadcasted_iota(jnp.int32, (N_EXPERTS, N_EXPERTS), 1)
    strict = jnp.where(r8 > c8, 1.0, 0.0)
    blkbase = lax.dot_general(strict, nblk, (((1,), (0,)), ((), ())),
                              preferred_element_type=jnp.float32)
    pos = blkbase * float(B) + csum - 1.0
    p1 = jnp.sum(jnp.where(sel1, pos, 0.0), axis=0, keepdims=True)
    p2 = jnp.sum(jnp.where(sel2, pos, 0.0), axis=0, keepdims=True)
    p1_ref[...] = jnp.reshape(p1, (T,)).astype(jnp.int32)
    p2_ref[...] = jnp.reshape(p2, (T,)).astype(jnp.int32)
    ii = lax.broadcasted_iota(jnp.int32, (1, 128), 1).astype(jnp.float32)
    acc = jnp.full((1, 128), -1.0, jnp.float32)
    for e in range(N_EXPERTS):
        acc = acc + jnp.where(blkbase[e:e + 1, 0:1] <= ii, 1.0, 0.0)
    eid_ref[...] = jnp.reshape(acc, (128,)).astype(jnp.int32)
    nu = jnp.sum(nblk, axis=0, keepdims=True)
    nu_ref[...] = jnp.reshape(nu, (1,)).astype(jnp.int32)


def _gate_meta(xf, gate_w):
    return pl.pallas_call(
        _gate_meta_body,
        grid=(1,),
        in_specs=[
            pl.BlockSpec((T, DIM), lambda i: (0, 0)),
            pl.BlockSpec((N_EXPERTS, DIM), lambda i: (0, 0)),
        ],
        out_specs=[
            pl.BlockSpec((T,), lambda i: (0,)),
            pl.BlockSpec((T,), lambda i: (0,)),
            pl.BlockSpec((T,), lambda i: (0,)),
            pl.BlockSpec((T,), lambda i: (0,)),
            pl.BlockSpec((128,), lambda i: (0,)),
            pl.BlockSpec((1,), lambda i: (0,)),
        ],
        out_shape=[
            jax.ShapeDtypeStruct((T,), jnp.int32),
            jax.ShapeDtypeStruct((T,), jnp.int32),
            jax.ShapeDtypeStruct((T,), jnp.float32),
            jax.ShapeDtypeStruct((T,), jnp.float32),
            jax.ShapeDtypeStruct((128,), jnp.int32),
            jax.ShapeDtypeStruct((1,), jnp.int32),
        ],
    )(xf, gate_w)


def _shared_body(x_ref, sw1_ref, sw2_ref, sw3_ref, ys_ref, xb_ref):
    xb = x_ref[...].astype(jnp.bfloat16)
    xb_ref[...] = xb
    w1 = sw1_ref[...].astype(jnp.bfloat16)
    w2 = sw2_ref[...].astype(jnp.bfloat16)
    w3 = sw3_ref[...].astype(jnp.bfloat16)
    h1 = lax.dot_general(xb, w1, (((1,), (1,)), ((), ())),
                         preferred_element_type=jnp.float32)
    h3 = lax.dot_general(xb, w3, (((1,), (1,)), ((), ())),
                         preferred_element_type=jnp.float32)
    h = (h1 * jax.nn.sigmoid(h1) * h3).astype(jnp.bfloat16)
    ys_ref[...] = lax.dot_general(h, w2, (((1,), (1,)), ((), ())),
                                  preferred_element_type=jnp.float32)


def _shared(xf, sw1, sw2, sw3):
    nb = T // TB
    return pl.pallas_call(
        _shared_body,
        grid=(nb,),
        in_specs=[
            pl.BlockSpec((TB, DIM), lambda i: (i, 0)),
            pl.BlockSpec((S_INTER, DIM), lambda i: (0, 0)),
            pl.BlockSpec((DIM, S_INTER), lambda i: (0, 0)),
            pl.BlockSpec((S_INTER, DIM), lambda i: (0, 0)),
        ],
        out_specs=[
            pl.BlockSpec((TB, DIM), lambda i: (i, 0)),
            pl.BlockSpec((TB, DIM), lambda i: (i, 0)),
        ],
        out_shape=[
            jax.ShapeDtypeStruct((T, DIM), jnp.float32),
            jax.ShapeDtypeStruct((T, DIM), jnp.bfloat16),
        ],
    )(xf, sw1, sw2, sw3)


def _sc_scatter_routing(pos1, pos2, wa, wb):
    def body(p1_hbm, p2_hbm, wa_hbm, wb_hbm, tok_hbm, wrow_hbm,
             tok_v, wrow_v, pos_v, w_v):
        @pl.when(_wid() == 0)
        def _():
            def init(i, carry):
                tok_v[pl.ds(i * 16, 16)] = jnp.zeros((16,), jnp.int32)
                wrow_v[pl.ds(i * 16, 16)] = jnp.zeros((16,), jnp.float32)
                return carry
            lax.fori_loop(0, NPAD // 16, init, 0)
            for p_hbm, wx_hbm in ((p1_hbm, wa_hbm), (p2_hbm, wb_hbm)):
                pltpu.sync_copy(p_hbm, pos_v)
                pltpu.sync_copy(wx_hbm, w_v)

                def step(i, carry):
                    idx = pos_v[pl.ds(i * 16, 16)]
                    tvals = lax.iota(jnp.int32, 16) + i * 16
                    plsc.store_scatter(tok_v, [idx], tvals)
                    wv = w_v[pl.ds(i * 16, 16)]
                    plsc.store_scatter(wrow_v, [idx], wv)
                    return carry
                lax.fori_loop(0, T // 16, step, 0)
            pltpu.sync_copy(tok_v, tok_hbm)
            pltpu.sync_copy(wrow_v, wrow_hbm)

    fn = pl.kernel(
        body,
        out_type=[
            jax.ShapeDtypeStruct((NPAD,), jnp.int32),
            jax.ShapeDtypeStruct((NPAD,), jnp.float32),
        ],
        mesh=_sc_mesh(),
        scratch_types=[
            pltpu.VMEM((NPAD,), jnp.int32),
            pltpu.VMEM((NPAD,), jnp.float32),
            pltpu.VMEM((T,), jnp.int32),
            pltpu.VMEM((T,), jnp.float32),
        ],
        compiler_params=pltpu.CompilerParams(needs_layout_passes=False),
    )
    return fn(pos1, pos2, wa, wb)


def _gemm_body(eid_ref, nu_ref, tok_ref, wrow_ref, xb_ref, ys_ref,
               ew1_ref, ew2_ref, ew3_ref, o_ref):
    i = pl.program_id(0)

    @pl.when(i == 0)
    def _():
        o_ref[...] = ys_ref[...]

    @pl.when(i < nu_ref[0])
    def _():
        tok = jnp.reshape(tok_ref[...], (1, B))
        wrow = jnp.reshape(wrow_ref[...], (1, B))
        tio = lax.broadcasted_iota(jnp.int32, (T, B), 0)
        ohm = tio == tok
        oh = ohm.astype(jnp.bfloat16)
        ohw = jnp.where(ohm, wrow, 0.0).astype(jnp.bfloat16)
        xg = lax.dot_general(oh, xb_ref[...], (((0,), (0,)), ((), ())),
                             preferred_element_type=jnp.float32
                             ).astype(jnp.bfloat16)
        w1 = ew1_ref[0].astype(jnp.bfloat16)
        w2 = ew2_ref[0].astype(jnp.bfloat16)
        w3 = ew3_ref[0].astype(jnp.bfloat16)
        h1 = lax.dot_general(xg, w1, (((1,), (1,)), ((), ())),
                             preferred_element_type=jnp.float32)
        h3 = lax.dot_general(xg, w3, (((1,), (1,)), ((), ())),
                             preferred_element_type=jnp.float32)
        h = (h1 * jax.nn.sigmoid(h1) * h3).astype(jnp.bfloat16)
        y = lax.dot_general(h, w2, (((1,), (1,)), ((), ())),
                            preferred_element_type=jnp.float32
                            ).astype(jnp.bfloat16)
        o_ref[...] += lax.dot_general(ohw, y, (((1,), (0,)), ((), ())),
                                      preferred_element_type=jnp.float32)


def _gemm(eid, nused, tok, wrow, xb, ys, ew1, ew2, ew3):
    return pl.pallas_call(
        _gemm_body,
        grid_spec=pltpu.PrefetchScalarGridSpec(
            num_scalar_prefetch=2,
            grid=(NB,),
            in_specs=[
                pl.BlockSpec((B,), lambda i, eid, nu: (i,)),
                pl.BlockSpec((B,), lambda i, eid, nu: (i,)),
                pl.BlockSpec((T, DIM), lambda i, eid, nu: (0, 0)),
                pl.BlockSpec((T, DIM), lambda i, eid, nu: (0, 0)),
                pl.BlockSpec((1, INTER, DIM), lambda i, eid, nu: (eid[i], 0, 0)),
                pl.BlockSpec((1, DIM, INTER), lambda i, eid, nu: (eid[i], 0, 0)),
                pl.BlockSpec((1, INTER, DIM), lambda i, eid, nu: (eid[i], 0, 0)),
            ],
            out_specs=pl.BlockSpec((T, DIM), lambda i, eid, nu: (0, 0)),
        ),
        out_shape=jax.ShapeDtypeStruct((T, DIM), jnp.float32),
    )(eid, nused, tok, wrow, xb, ys, ew1, ew2, ew3)


def kernel(x, gate_w, ew1, ew2, ew3, sw1, sw2, sw3):
    shape = x.shape
    xf = x.reshape(-1, DIM)

    pos1, pos2, wa, wb, eid, nused = _gate_meta(xf, gate_w)
    tok, wrow = _sc_scatter_routing(pos1, pos2, wa, wb)
    ys, xb = _shared(xf, sw1, sw2, sw3)
    y = _gemm(eid, nused, tok, wrow, xb, ys, ew1, ew2, ew3)
    return y.reshape(shape)
